# baseline (speedup 1.0000x reference)
import functools

import jax
import jax.numpy as jnp
from jax import lax
from jax.experimental import pallas as pl
from jax.experimental.pallas import tpu as pltpu
from jax.experimental.pallas import tpu_sc as plsc


def _make_sc_gather(B, UD, AD, CH):
    info = plsc.get_sparse_core_info()
    NC, NS = info.num_cores, info.num_subcores
    NW = NC * NS
    b_per_w = B // NW
    n_chunks = b_per_w // CH
    mesh = plsc.VectorSubcoreMesh(core_axis_name="c", subcore_axis_name="s")

    @functools.partial(
        pl.kernel,
        mesh=mesh,
        out_type=(
            jax.ShapeDtypeStruct((B, UD), jnp.float32),
            jax.ShapeDtypeStruct((B, AD), jnp.float32),
        ),
        scratch_types=[
            pltpu.VMEM((b_per_w,), jnp.int32),
            pltpu.VMEM((CH,), jnp.int32),
            pltpu.VMEM((CH, UD), jnp.float32),
            pltpu.VMEM((CH, AD), jnp.float32),
            pltpu.SemaphoreType.DMA,
            pltpu.SemaphoreType.DMA,
        ],
    )
    def sc_gather(user_hbm, song_hbm, utab_hbm, atab_hbm, uout_hbm, aout_hbm,
                  uidx_v, aidx_v, urows_v, arows_v, sem_u, sem_a):
        wid = lax.axis_index("s") * NC + lax.axis_index("c")
        base = wid * b_per_w
        pltpu.sync_copy(user_hbm.at[pl.ds(base, b_per_w)], uidx_v)
        for k in range(n_chunks):
            off = base + k * CH
            pltpu.sync_copy(song_hbm.at[pl.ds(off, CH)], aidx_v)
            cp_a = pltpu.async_copy(atab_hbm.at[aidx_v], arows_v, sem_a)
            for g in range(CH // 16):
                uvec = uidx_v[pl.ds(k * CH + g * 16, 16)]
                for j in range(16):
                    pltpu.async_copy(
                        utab_hbm.at[pl.ds(uvec[j], 1)],
                        urows_v.at[pl.ds(g * 16 + j, 1)], sem_u)
            pltpu.make_async_copy(
                utab_hbm.at[pl.ds(0, CH)], urows_v, sem_u).wait()
            pltpu.sync_copy(urows_v, uout_hbm.at[pl.ds(off, CH)])
            cp_a.wait()
            pltpu.sync_copy(arows_v, aout_hbm.at[pl.ds(off, CH)])

    return sc_gather


def _mlp_body(xu_ref, xa_ref, w1u_ref, w1a_ref, b1_ref, w2_ref, b2_ref,
              w3_ref, b3_ref, out_ref):
    h = jnp.dot(xu_ref[...], w1u_ref[...], preferred_element_type=jnp.float32)
    h = h + jnp.dot(xa_ref[...], w1a_ref[...],
                    preferred_element_type=jnp.float32)
    h = jnp.maximum(h + b1_ref[...], 0.0)
    h = jnp.dot(h, w2_ref[...], preferred_element_type=jnp.float32)
    h = jnp.maximum(h + b2_ref[...], 0.0)
    o = jnp.dot(h, w3_ref[...], preferred_element_type=jnp.float32)
    o = o + b3_ref[...]
    out_ref[...] = 1.0 / (1.0 + jnp.exp(-o))


def _mlp(xu, xa, W1u, W1a, b1, W2, b2, W3, b3, bm):
    B, UD = xu.shape
    AD = xa.shape[1]
    H1 = W1u.shape[1]
    H2 = W2.shape[1]
    grid = (B // bm,)
    return pl.pallas_call(
        _mlp_body,
        grid=grid,
        in_specs=[
            pl.BlockSpec((bm, UD), lambda i: (i, 0)),
            pl.BlockSpec((bm, AD), lambda i: (i, 0)),
            pl.BlockSpec((UD, H1), lambda i: (0, 0)),
            pl.BlockSpec((AD, H1), lambda i: (0, 0)),
            pl.BlockSpec((1, H1), lambda i: (0, 0)),
            pl.BlockSpec((H1, H2), lambda i: (0, 0)),
            pl.BlockSpec((1, H2), lambda i: (0, 0)),
            pl.BlockSpec((H2, 1), lambda i: (0, 0)),
            pl.BlockSpec((1, 1), lambda i: (0, 0)),
        ],
        out_specs=pl.BlockSpec((bm, 1), lambda i: (i, 0)),
        out_shape=jax.ShapeDtypeStruct((B, 1), jnp.float32),
    )(xu, xa, W1u, W1a, b1, W2, b2, W3, b3)


@jax.jit
def kernel(user, song, user_table, audio_table, W1, b1, W2, b2, W3, b3):
    B = user.shape[0]
    UD = user_table.shape[1]
    AD = audio_table.shape[1]

    NCH = 8
    Bc = B // NCH
    sc_gather = _make_sc_gather(Bc, UD, AD, CH=64)
    W1u = W1[:UD]
    W1a = W1[UD:]
    b1r = b1.reshape(1, -1)
    b2r = b2.reshape(1, -1)
    b3r = b3.reshape(1, 1)
    outs = []
    for c in range(NCH):
        ue, ae = sc_gather(user[c * Bc:(c + 1) * Bc], song[c * Bc:(c + 1) * Bc],
                           user_table, audio_table)
        outs.append(_mlp(ue, ae, W1u, W1a, b1r, W2, b2r, W3, b3r, bm=512))
    return jnp.concatenate(outs, axis=0)


# speedup vs baseline: 1.0265x; 1.0265x over previous
import functools

import jax
import jax.numpy as jnp
from jax import lax
from jax.experimental import pallas as pl
from jax.experimental.pallas import tpu as pltpu
from jax.experimental.pallas import tpu_sc as plsc


def _make_sc_gather(B, UD, AD, CH):
    info = plsc.get_sparse_core_info()
    NC, NS = info.num_cores, info.num_subcores
    NW = NC * NS
    b_per_w = B // NW
    n_chunks = b_per_w // CH
    mesh = plsc.VectorSubcoreMesh(core_axis_name="c", subcore_axis_name="s")

    @functools.partial(
        pl.kernel,
        mesh=mesh,
        out_type=(
            jax.ShapeDtypeStruct((B, UD), jnp.float32),
            jax.ShapeDtypeStruct((B, AD), jnp.float32),
        ),
        scratch_types=[
            pltpu.VMEM((b_per_w,), jnp.int32),
            pltpu.VMEM((b_per_w,), jnp.int32),
            pltpu.VMEM((CH, UD), jnp.float32),
            pltpu.VMEM((CH, AD), jnp.float32),
            pltpu.VMEM((CH, AD), jnp.float32),
            pltpu.SemaphoreType.DMA,
            pltpu.SemaphoreType.DMA,
        ],
    )
    def sc_gather(user_hbm, song_hbm, utab_hbm, atab_hbm, uout_hbm, aout_hbm,
                  uidx_v, aidx_v, urows_v, arows0_v, arows1_v, sem_u, sem_a):
        wid = lax.axis_index("s") * NC + lax.axis_index("c")
        base = wid * b_per_w
        pltpu.sync_copy(user_hbm.at[pl.ds(base, b_per_w)], uidx_v)
        pltpu.sync_copy(song_hbm.at[pl.ds(base, b_per_w)], aidx_v)
        abufs = [arows0_v, arows1_v]
        acps = [None, None]
        acps[0] = pltpu.async_copy(
            atab_hbm.at[aidx_v.at[pl.ds(0, CH)]], abufs[0], sem_a)
        for k in range(n_chunks):
            if k + 1 < n_chunks:
                acps[(k + 1) % 2] = pltpu.async_copy(
                    atab_hbm.at[aidx_v.at[pl.ds((k + 1) * CH, CH)]],
                    abufs[(k + 1) % 2], sem_a)
            for g in range(CH // 16):
                uvec = uidx_v[pl.ds(k * CH + g * 16, 16)]
                for j in range(16):
                    pltpu.async_copy(
                        utab_hbm.at[pl.ds(uvec[j], 1)],
                        urows_v.at[pl.ds(g * 16 + j, 1)], sem_u)
            pltpu.make_async_copy(
                utab_hbm.at[pl.ds(0, CH)], urows_v, sem_u).wait()
            pltpu.sync_copy(urows_v, uout_hbm.at[pl.ds(base + k * CH, CH)])
            acps[k % 2].wait()
            pltpu.sync_copy(abufs[k % 2], aout_hbm.at[pl.ds(base + k * CH, CH)])

    return sc_gather


def _mlp_body(xu_ref, xa_ref, w1u_ref, w1a_ref, b1_ref, w2_ref, b2_ref,
              w3_ref, b3_ref, out_ref):
    h = jnp.dot(xu_ref[...], w1u_ref[...], preferred_element_type=jnp.float32)
    h = h + jnp.dot(xa_ref[...], w1a_ref[...],
                    preferred_element_type=jnp.float32)
    h = jnp.maximum(h + b1_ref[...], 0.0)
    h = jnp.dot(h, w2_ref[...], preferred_element_type=jnp.float32)
    h = jnp.maximum(h + b2_ref[...], 0.0)
    o = jnp.dot(h, w3_ref[...], preferred_element_type=jnp.float32)
    o = o + b3_ref[...]
    out_ref[...] = 1.0 / (1.0 + jnp.exp(-o))


def _mlp(xu, xa, W1u, W1a, b1, W2, b2, W3, b3, bm):
    B, UD = xu.shape
    AD = xa.shape[1]
    H1 = W1u.shape[1]
    H2 = W2.shape[1]
    grid = (B // bm,)
    return pl.pallas_call(
        _mlp_body,
        grid=grid,
        in_specs=[
            pl.BlockSpec((bm, UD), lambda i: (i, 0)),
            pl.BlockSpec((bm, AD), lambda i: (i, 0)),
            pl.BlockSpec((UD, H1), lambda i: (0, 0)),
            pl.BlockSpec((AD, H1), lambda i: (0, 0)),
            pl.BlockSpec((1, H1), lambda i: (0, 0)),
            pl.BlockSpec((H1, H2), lambda i: (0, 0)),
            pl.BlockSpec((1, H2), lambda i: (0, 0)),
            pl.BlockSpec((H2, 1), lambda i: (0, 0)),
            pl.BlockSpec((1, 1), lambda i: (0, 0)),
        ],
        out_specs=pl.BlockSpec((bm, 1), lambda i: (i, 0)),
        out_shape=jax.ShapeDtypeStruct((B, 1), jnp.float32),
    )(xu, xa, W1u, W1a, b1, W2, b2, W3, b3)


@jax.jit
def kernel(user, song, user_table, audio_table, W1, b1, W2, b2, W3, b3):
    B = user.shape[0]
    UD = user_table.shape[1]
    AD = audio_table.shape[1]

    NCH = 4
    Bc = B // NCH
    sc_gather = _make_sc_gather(Bc, UD, AD, CH=64)
    W1u = W1[:UD]
    W1a = W1[UD:]
    b1r = b1.reshape(1, -1)
    b2r = b2.reshape(1, -1)
    b3r = b3.reshape(1, 1)
    outs = []
    for c in range(NCH):
        ue, ae = sc_gather(user[c * Bc:(c + 1) * Bc], song[c * Bc:(c + 1) * Bc],
                           user_table, audio_table)
        outs.append(_mlp(ue, ae, W1u, W1a, b1r, W2, b2r, W3, b3r, bm=512))
    return jnp.concatenate(outs, axis=0)


# speedup vs baseline: 1.0409x; 1.0141x over previous
import functools

import jax
import jax.numpy as jnp
from jax import lax
from jax.experimental import pallas as pl
from jax.experimental.pallas import tpu as pltpu
from jax.experimental.pallas import tpu_sc as plsc


def _make_sc_gather(B, UD, AD, CH):
    info = plsc.get_sparse_core_info()
    NC, NS = info.num_cores, info.num_subcores
    NW = NC * NS
    b_per_w = B // NW
    n_chunks = b_per_w // CH
    mesh = plsc.VectorSubcoreMesh(core_axis_name="c", subcore_axis_name="s")

    @functools.partial(
        pl.kernel,
        mesh=mesh,
        out_type=(
            jax.ShapeDtypeStruct((B, UD), jnp.float32),
            jax.ShapeDtypeStruct((B, AD), jnp.float32),
        ),
        scratch_types=[
            pltpu.VMEM((b_per_w,), jnp.int32),
            pltpu.VMEM((CH,), jnp.int32),
            pltpu.VMEM((CH, UD), jnp.float32),
            pltpu.VMEM((CH, AD), jnp.float32),
            pltpu.SemaphoreType.DMA,
            pltpu.SemaphoreType.DMA,
        ],
    )
    def sc_gather(user_hbm, song_hbm, utab_hbm, atab_hbm, uout_hbm, aout_hbm,
                  uidx_v, aidx_v, urows_v, arows_v, sem_u, sem_a):
        wid = lax.axis_index("s") * NC + lax.axis_index("c")
        base = wid * b_per_w
        pltpu.sync_copy(user_hbm.at[pl.ds(base, b_per_w)], uidx_v)
        for k in range(n_chunks):
            off = base + k * CH
            pltpu.sync_copy(song_hbm.at[pl.ds(off, CH)], aidx_v)
            cp_a = pltpu.async_copy(atab_hbm.at[aidx_v], arows_v, sem_a)
            for g in range(CH // 16):
                uvec = uidx_v[pl.ds(k * CH + g * 16, 16)]
                for j in range(16):
                    pltpu.async_copy(
                        utab_hbm.at[pl.ds(uvec[j], 1)],
                        urows_v.at[pl.ds(g * 16 + j, 1)], sem_u)
            pltpu.make_async_copy(
                utab_hbm.at[pl.ds(0, CH)], urows_v, sem_u).wait()
            pltpu.sync_copy(urows_v, uout_hbm.at[pl.ds(off, CH)])
            cp_a.wait()
            pltpu.sync_copy(arows_v, aout_hbm.at[pl.ds(off, CH)])

    return sc_gather


def _mlp_body(xu_ref, xa_ref, w1u_ref, w1a_ref, b1_ref, w2_ref, b2_ref,
              w3_ref, b3_ref, out_ref):
    h = jnp.dot(xu_ref[...], w1u_ref[...], preferred_element_type=jnp.float32)
    h = h + jnp.dot(xa_ref[...], w1a_ref[...],
                    preferred_element_type=jnp.float32)
    h = jnp.maximum(h + b1_ref[...], 0.0)
    h = jnp.dot(h, w2_ref[...], preferred_element_type=jnp.float32)
    h = jnp.maximum(h + b2_ref[...], 0.0)
    o = jnp.dot(h, w3_ref[...], preferred_element_type=jnp.float32)
    o = o + b3_ref[...]
    out_ref[...] = 1.0 / (1.0 + jnp.exp(-o))


def _mlp(xu, xa, W1u, W1a, b1, W2, b2, W3, b3, bm):
    B, UD = xu.shape
    AD = xa.shape[1]
    H1 = W1u.shape[1]
    H2 = W2.shape[1]
    grid = (B // bm,)
    return pl.pallas_call(
        _mlp_body,
        grid=grid,
        in_specs=[
            pl.BlockSpec((bm, UD), lambda i: (i, 0)),
            pl.BlockSpec((bm, AD), lambda i: (i, 0)),
            pl.BlockSpec((UD, H1), lambda i: (0, 0)),
            pl.BlockSpec((AD, H1), lambda i: (0, 0)),
            pl.BlockSpec((1, H1), lambda i: (0, 0)),
            pl.BlockSpec((H1, H2), lambda i: (0, 0)),
            pl.BlockSpec((1, H2), lambda i: (0, 0)),
            pl.BlockSpec((H2, 1), lambda i: (0, 0)),
            pl.BlockSpec((1, 1), lambda i: (0, 0)),
        ],
        out_specs=pl.BlockSpec((bm, 1), lambda i: (i, 0)),
        out_shape=jax.ShapeDtypeStruct((B, 1), jnp.float32),
    )(xu, xa, W1u, W1a, b1, W2, b2, W3, b3)


@jax.jit
def kernel(user, song, user_table, audio_table, W1, b1, W2, b2, W3, b3):
    B = user.shape[0]
    UD = user_table.shape[1]
    AD = audio_table.shape[1]

    NCH = 4
    Bc = B // NCH
    sc_gather = _make_sc_gather(Bc, UD, AD, CH=128)
    W1u = W1[:UD]
    W1a = W1[UD:]
    b1r = b1.reshape(1, -1)
    b2r = b2.reshape(1, -1)
    b3r = b3.reshape(1, 1)
    outs = []
    for c in range(NCH):
        ue, ae = sc_gather(user[c * Bc:(c + 1) * Bc], song[c * Bc:(c + 1) * Bc],
                           user_table, audio_table)
        outs.append(_mlp(ue, ae, W1u, W1a, b1r, W2, b2r, W3, b3r, bm=1024))
    return jnp.concatenate(outs, axis=0)


# speedup vs baseline: 1.0478x; 1.0065x over previous
import functools

import jax
import jax.numpy as jnp
from jax import lax
from jax.experimental import pallas as pl
from jax.experimental.pallas import tpu as pltpu
from jax.experimental.pallas import tpu_sc as plsc


def _make_sc_gather(B, UD, AD, CH):
    info = plsc.get_sparse_core_info()
    NC, NS = info.num_cores, info.num_subcores
    NW = NC * NS
    b_per_w = B // NW
    n_chunks = b_per_w // CH
    mesh = plsc.VectorSubcoreMesh(core_axis_name="c", subcore_axis_name="s")

    @functools.partial(
        pl.kernel,
        mesh=mesh,
        out_type=(
            jax.ShapeDtypeStruct((B, UD), jnp.float32),
            jax.ShapeDtypeStruct((B, AD), jnp.float32),
        ),
        scratch_types=[
            pltpu.VMEM((b_per_w,), jnp.int32),
            pltpu.VMEM((CH,), jnp.int32),
            pltpu.VMEM((CH, UD), jnp.float32),
            pltpu.VMEM((CH, AD), jnp.float32),
            pltpu.SemaphoreType.DMA,
            pltpu.SemaphoreType.DMA,
        ],
    )
    def sc_gather(user_hbm, song_hbm, utab_hbm, atab_hbm, uout_hbm, aout_hbm,
                  uidx_v, aidx_v, urows_v, arows_v, sem_u, sem_a):
        wid = lax.axis_index("s") * NC + lax.axis_index("c")
        base = wid * b_per_w
        pltpu.sync_copy(user_hbm.at[pl.ds(base, b_per_w)], uidx_v)
        for k in range(n_chunks):
            off = base + k * CH
            pltpu.sync_copy(song_hbm.at[pl.ds(off, CH)], aidx_v)
            cp_a = pltpu.async_copy(atab_hbm.at[aidx_v], arows_v, sem_a)
            for g in range(CH // 16):
                uvec = uidx_v[pl.ds(k * CH + g * 16, 16)]
                for j in range(16):
                    pltpu.async_copy(
                        utab_hbm.at[pl.ds(uvec[j], 1)],
                        urows_v.at[pl.ds(g * 16 + j, 1)], sem_u)
            pltpu.make_async_copy(
                utab_hbm.at[pl.ds(0, CH)], urows_v, sem_u).wait()
            pltpu.sync_copy(urows_v, uout_hbm.at[pl.ds(off, CH)])
            cp_a.wait()
            pltpu.sync_copy(arows_v, aout_hbm.at[pl.ds(off, CH)])

    return sc_gather


def _mlp_body(xu_ref, xa_ref, w1u_ref, w1a_ref, b1_ref, w2_ref, b2_ref,
              w3_ref, b3_ref, out_ref):
    h = jnp.dot(xu_ref[...], w1u_ref[...], preferred_element_type=jnp.float32)
    h = h + jnp.dot(xa_ref[...], w1a_ref[...],
                    preferred_element_type=jnp.float32)
    h = jnp.maximum(h + b1_ref[...], 0.0)
    h = jnp.dot(h, w2_ref[...], preferred_element_type=jnp.float32)
    h = jnp.maximum(h + b2_ref[...], 0.0)
    o = jnp.dot(h, w3_ref[...], preferred_element_type=jnp.float32)
    o = o + b3_ref[...]
    out_ref[...] = 1.0 / (1.0 + jnp.exp(-o))


def _mlp(xu, xa, W1u, W1a, b1, W2, b2, W3, b3, bm):
    B, UD = xu.shape
    AD = xa.shape[1]
    H1 = W1u.shape[1]
    H2 = W2.shape[1]
    grid = (B // bm,)
    return pl.pallas_call(
        _mlp_body,
        grid=grid,
        in_specs=[
            pl.BlockSpec((bm, UD), lambda i: (i, 0)),
            pl.BlockSpec((bm, AD), lambda i: (i, 0)),
            pl.BlockSpec((UD, H1), lambda i: (0, 0)),
            pl.BlockSpec((AD, H1), lambda i: (0, 0)),
            pl.BlockSpec((1, H1), lambda i: (0, 0)),
            pl.BlockSpec((H1, H2), lambda i: (0, 0)),
            pl.BlockSpec((1, H2), lambda i: (0, 0)),
            pl.BlockSpec((H2, 1), lambda i: (0, 0)),
            pl.BlockSpec((1, 1), lambda i: (0, 0)),
        ],
        out_specs=pl.BlockSpec((bm, 1), lambda i: (i, 0)),
        out_shape=jax.ShapeDtypeStruct((B, 1), jnp.float32),
    )(xu, xa, W1u, W1a, b1, W2, b2, W3, b3)


@jax.jit
def kernel(user, song, user_table, audio_table, W1, b1, W2, b2, W3, b3):
    B = user.shape[0]
    UD = user_table.shape[1]
    AD = audio_table.shape[1]

    NCH = 4
    Bc = B // NCH
    sc_gather = _make_sc_gather(Bc, UD, AD, CH=128)
    W1u = W1[:UD]
    W1a = W1[UD:]
    b1r = b1.reshape(1, -1)
    b2r = b2.reshape(1, -1)
    b3r = b3.reshape(1, 1)
    outs = []
    for c in range(NCH):
        ue, ae = sc_gather(user[c * Bc:(c + 1) * Bc], song[c * Bc:(c + 1) * Bc],
                           user_table, audio_table)
        outs.append(_mlp(ue, ae, W1u, W1a, b1r, W2, b2r, W3, b3r, bm=2048))
    return jnp.concatenate(outs, axis=0)
